# Initial kernel scaffold; baseline (speedup 1.0000x reference)
import jax
import jax.numpy as jnp
from jax.experimental import pallas as pl


def kernel(x, time_table, weekday_table, W1, b1, g1, be1, W2, b2, g2, be2, W3, b3, g3, be3, W4, b4):
    raise NotImplementedError("write your pallas kernel here")


# speedup vs baseline: 3.3215x; 3.3215x over previous
import jax
import jax.numpy as jnp
from jax.experimental import pallas as pl

_N = 16384
_EPS = 1e-5


def _body(x_ref, tt_ref, wt_ref, w1t_ref, b1_ref, g1_ref, be1_ref,
          w2t_ref, b2_ref, g2_ref, be2_ref, w3t_ref, b3_ref, g3_ref,
          be3_ref, w4t_ref, b4_ref, out_ref):
    f32 = jnp.float32
    n = x_ref.shape[0]
    xv = x_ref[...]
    tb = jnp.clip(xv[:, 0:1], 0.0, 3.0)
    wd = jnp.clip(xv[:, 1:2], 0.0, 6.0)
    idx = (tb * 7.0 + wd).astype(jnp.int32)
    lane = jax.lax.broadcasted_iota(jnp.int32, (n, 32), 1)
    onehot = (idx == lane).astype(f32)
    counts = jnp.sum(onehot, axis=0, keepdims=True)

    krow = jax.lax.broadcasted_iota(jnp.int32, (32, 8), 0)
    col8 = jax.lax.broadcasted_iota(jnp.int32, (32, 8), 1)
    oh_tb = (krow // 7 == col8).astype(f32)
    oh_wd = (krow % 7 == col8).astype(f32)
    hi = jax.lax.Precision.HIGHEST
    emb_t = jnp.dot(oh_tb, tt_ref[...], preferred_element_type=f32, precision=hi)
    emb_w = jnp.dot(oh_wd, wt_ref[...], preferred_element_type=f32, precision=hi)
    emb = jnp.concatenate([emb_t, emb_w], axis=1)
    h = jnp.dot(emb, w1t_ref[...], preferred_element_type=f32) + b1_ref[...]

    inv_n = 1.0 / n

    def bn_relu(ht, g, be):
        m = jnp.dot(counts, ht, preferred_element_type=f32, precision=hi) * inv_n
        d = ht - m
        v = jnp.dot(counts, d * d, preferred_element_type=f32, precision=hi) * inv_n
        return jnp.maximum(g * d / jnp.sqrt(v + _EPS) + be, 0.0)

    h = bn_relu(h, g1_ref[...], be1_ref[...])
    h = jnp.dot(h, w2t_ref[...], preferred_element_type=f32) + b2_ref[...]
    h = bn_relu(h, g2_ref[...], be2_ref[...])
    h = jnp.dot(h, w3t_ref[...], preferred_element_type=f32) + b3_ref[...]
    h = bn_relu(h, g3_ref[...], be3_ref[...])
    table = jnp.dot(h, w4t_ref[...], preferred_element_type=f32) + b4_ref[...]
    out_ref[...] = jnp.dot(onehot, table, preferred_element_type=f32, precision=hi)


def kernel(x, time_table, weekday_table, W1, b1, g1, be1, W2, b2, g2, be2,
           W3, b3, g3, be3, W4, b4):
    f32 = jnp.float32
    tt8 = jnp.zeros((8, 8), f32).at[:4].set(time_table)
    wt8 = jnp.zeros((8, 8), f32).at[:7].set(weekday_table)
    args = (
        x, tt8, wt8,
        W1.T, b1.reshape(1, 32), g1.reshape(1, 32), be1.reshape(1, 32),
        W2.T, b2.reshape(1, 16), g2.reshape(1, 16), be2.reshape(1, 16),
        W3.T, b3.reshape(1, 8), g3.reshape(1, 8), be3.reshape(1, 8),
        W4.T, b4.reshape(1, 1),
    )
    return pl.pallas_call(
        _body,
        out_shape=jax.ShapeDtypeStruct((_N, 1), f32),
    )(*args)


# speedup vs baseline: 4.2086x; 1.2671x over previous
import functools

import jax
import jax.numpy as jnp
from jax import lax
from jax.experimental import pallas as pl
from jax.experimental.pallas import tpu as pltpu
from jax.experimental.pallas import tpu_sc as plsc

_N = 16384
_EPS = 1e-5
_NW = 32
_CHUNK = _N // _NW


def _tc_body(x0_ref, x1_ref, tt_ref, wt_ref, w1t_ref, b1_ref, g1_ref,
             be1_ref, w2t_ref, b2_ref, g2_ref, be2_ref, w3t_ref, b3_ref,
             g3_ref, be3_ref, w4t_ref, b4_ref, tab_ref):
    f32 = jnp.float32
    hi = lax.Precision.HIGHEST
    tb = jnp.clip(x0_ref[...], 0.0, 3.0)
    wd = jnp.clip(x1_ref[...], 0.0, 6.0)
    idx2d = (tb * 7.0 + wd).astype(jnp.int32)

    lane32 = lax.broadcasted_iota(jnp.int32, (1, 32), 1)
    counts = jnp.zeros((1, 32), f32)
    for k in range(28):
        ck = jnp.sum((idx2d == k).astype(f32))
        counts = counts + jnp.where(lane32 == k, ck, 0.0)

    krow = lax.broadcasted_iota(jnp.int32, (32, 8), 0)
    col8 = lax.broadcasted_iota(jnp.int32, (32, 8), 1)
    oh_tb = (krow // 7 == col8).astype(f32)
    oh_wd = (krow % 7 == col8).astype(f32)
    emb_t = jnp.dot(oh_tb, tt_ref[...], preferred_element_type=f32, precision=hi)
    emb_w = jnp.dot(oh_wd, wt_ref[...], preferred_element_type=f32, precision=hi)
    emb = jnp.concatenate([emb_t, emb_w], axis=1)
    h = jnp.dot(emb, w1t_ref[...], preferred_element_type=f32) + b1_ref[...]

    inv_n = 1.0 / _N

    def bn_relu(ht, g, be):
        m = jnp.dot(counts, ht, preferred_element_type=f32, precision=hi) * inv_n
        d = ht - m
        v = jnp.dot(counts, d * d, preferred_element_type=f32, precision=hi) * inv_n
        return jnp.maximum(g * d / jnp.sqrt(v + _EPS) + be, 0.0)

    h = bn_relu(h, g1_ref[...], be1_ref[...])
    h = jnp.dot(h, w2t_ref[...], preferred_element_type=f32) + b2_ref[...]
    h = bn_relu(h, g2_ref[...], be2_ref[...])
    h = jnp.dot(h, w3t_ref[...], preferred_element_type=f32) + b3_ref[...]
    h = bn_relu(h, g3_ref[...], be3_ref[...])
    tab_ref[...] = jnp.dot(h, w4t_ref[...], preferred_element_type=f32) + b4_ref[...]


_SC_MESH = plsc.VectorSubcoreMesh(core_axis_name="c", subcore_axis_name="s")


@functools.partial(
    pl.kernel,
    out_type=jax.ShapeDtypeStruct((_N,), jnp.float32),
    mesh=_SC_MESH,
    compiler_params=pltpu.CompilerParams(needs_layout_passes=False),
    scratch_types=[
        pltpu.VMEM((_CHUNK,), jnp.float32),
        pltpu.VMEM((_CHUNK,), jnp.float32),
        pltpu.VMEM((32,), jnp.float32),
        pltpu.VMEM((_CHUNK,), jnp.float32),
    ],
)
def _sc_gather(x0_hbm, x1_hbm, tab_hbm, out_hbm, x0_v, x1_v, tab_v, out_v):
    wid = lax.axis_index("s") * 2 + lax.axis_index("c")
    base = wid * _CHUNK
    pltpu.sync_copy(x0_hbm.at[pl.ds(base, _CHUNK)], x0_v)
    pltpu.sync_copy(x1_hbm.at[pl.ds(base, _CHUNK)], x1_v)
    pltpu.sync_copy(tab_hbm, tab_v)

    def body(i, carry):
        a = jnp.clip(x0_v[pl.ds(i * 16, 16)], 0.0, 3.0)
        b = jnp.clip(x1_v[pl.ds(i * 16, 16)], 0.0, 6.0)
        idx = (a * 7.0 + b).astype(jnp.int32)
        out_v[pl.ds(i * 16, 16)] = plsc.load_gather(tab_v, [idx])
        return carry

    lax.fori_loop(0, _CHUNK // 16, body, 0)
    pltpu.sync_copy(out_v, out_hbm.at[pl.ds(base, _CHUNK)])


def kernel(x, time_table, weekday_table, W1, b1, g1, be1, W2, b2, g2, be2,
           W3, b3, g3, be3, W4, b4):
    f32 = jnp.float32
    x0 = x[:, 0]
    x1 = x[:, 1]
    tt8 = jnp.zeros((8, 8), f32).at[:4].set(time_table)
    wt8 = jnp.zeros((8, 8), f32).at[:7].set(weekday_table)
    tab = pl.pallas_call(
        _tc_body,
        out_shape=jax.ShapeDtypeStruct((32, 1), f32),
    )(
        x0.reshape(128, 128), x1.reshape(128, 128), tt8, wt8,
        W1.T, b1.reshape(1, 32), g1.reshape(1, 32), be1.reshape(1, 32),
        W2.T, b2.reshape(1, 16), g2.reshape(1, 16), be2.reshape(1, 16),
        W3.T, b3.reshape(1, 8), g3.reshape(1, 8), be3.reshape(1, 8),
        W4.T, b4.reshape(1, 1),
    )
    out = _sc_gather(x0, x1, tab.reshape(32))
    return out.reshape(_N, 1)
